# baseline (device time: 34876 ns/iter reference)
import jax
import jax.numpy as jnp
from jax import lax
from jax.experimental import pallas as pl
from jax.experimental.pallas import tpu as pltpu

N_DEV = 4
N_PHASES = 6
RECV_ORDER = (1, 3, 2)
SEND_ORDER = (2, 1, 3)


def kernel(x, Win0, Wout0, Win1, Wout1, Win2, Wout2):
    b, d = x.shape
    h = Win0.shape[1]

    def body(x_hbm, win0, wout0, win1, wout1, win2, wout2, out_hbm,
             xv, win_v, wout_v, outv,
             xbuf0, xbuf1, xbuf2, pbuf0, pbuf1, pbuf2,
             psend0, psend1, psend2, copy_sems, send_sems, recv_sems):
        me = lax.axis_index("i")
        xbufs = [xbuf0, xbuf1, xbuf2]
        pbufs = [pbuf0, pbuf1, pbuf2]
        psends = [psend0, psend1, psend2]
        sends = []

        cp_x = pltpu.make_async_copy(x_hbm, xv, copy_sems.at[0])
        cp_x.start()
        wcopies = []
        for l, (wi, wo) in enumerate([(win0, wout0), (win1, wout1), (win2, wout2)]):
            c_in = pltpu.make_async_copy(wi, win_v.at[l], copy_sems.at[1 + 2 * l])
            c_out = pltpu.make_async_copy(wo, wout_v.at[l], copy_sems.at[2 + 2 * l])
            c_in.start()
            c_out.start()
            wcopies.append((c_in, c_out))

        def send(phase, idx, src, dst_buf, tgt):
            rdma = pltpu.make_async_remote_copy(
                src_ref=src,
                dst_ref=dst_buf.at[me],
                send_sem=send_sems.at[phase, idx],
                recv_sem=recv_sems.at[phase, idx],
                device_id=(tgt,),
                device_id_type=pl.DeviceIdType.MESH,
            )
            rdma.start()
            sends.append(rdma)

        def wait_recv(phase, idx, buf):
            rdma = pltpu.make_async_remote_copy(
                src_ref=buf.at[me],
                dst_ref=buf.at[me],
                send_sem=send_sems.at[phase, idx],
                recv_sem=recv_sems.at[phase, idx],
                device_id=(me,),
                device_id_type=pl.DeviceIdType.MESH,
            )
            rdma.wait_recv()

        def mlp(xc, l):
            hh = jnp.maximum(
                jnp.dot(xc, win_v[l], preferred_element_type=jnp.float32), 0.0
            )
            return jnp.dot(hh, wout_v[l], preferred_element_type=jnp.float32)

        cp_x.wait()
        xbuf0[me] = xv[...]
        for off in SEND_ORDER:
            send(0, off - 1, xv, xbuf0, (me + off) % N_DEV)

        for l in range(3):
            wcopies[l][0].wait()
            wcopies[l][1].wait()
            gphase = 2 * l
            sphase = 2 * l + 1
            pbufs[l][me] = mlp(xbufs[l][me], l)
            for off in RECV_ORDER:
                src = (me - off) % N_DEV
                wait_recv(gphase, off - 1, xbufs[l])
                psends[l][src] = mlp(xbufs[l][src], l)
                send(sphase, 3 - off, psends[l].at[src], pbufs[l], src)
            for off in RECV_ORDER:
                wait_recv(sphase, off - 1, pbufs[l])
            pv = pbufs[l][...]
            reduced = pv[0] + pv[1] + pv[2] + pv[3]
            if l < 2:
                xbufs[l + 1][me] = reduced
                for off in SEND_ORDER:
                    send(2 * l + 2, off - 1, xbufs[l + 1].at[me],
                         xbufs[l + 1], (me + off) % N_DEV)
            else:
                outv[...] = reduced

        cp_out = pltpu.make_async_copy(outv, out_hbm, copy_sems.at[7])
        cp_out.start()
        for rdma in sends:
            rdma.wait_send()
        cp_out.wait()

    return pl.pallas_call(
        body,
        out_shape=jax.ShapeDtypeStruct((b, d), jnp.float32),
        in_specs=[pl.BlockSpec(memory_space=pl.ANY)] * 7,
        out_specs=pl.BlockSpec(memory_space=pl.ANY),
        scratch_shapes=(
            [
                pltpu.VMEM((b, d), jnp.float32),
                pltpu.VMEM((3, d, h), jnp.float32),
                pltpu.VMEM((3, h, d), jnp.float32),
                pltpu.VMEM((b, d), jnp.float32),
            ]
            + [pltpu.VMEM((N_DEV, b, d), jnp.float32)] * 3
            + [pltpu.VMEM((N_DEV, b, d), jnp.float32)] * 3
            + [pltpu.VMEM((N_DEV, b, d), jnp.float32)] * 3
            + [
                pltpu.SemaphoreType.DMA((8,)),
                pltpu.SemaphoreType.DMA((N_PHASES, N_DEV - 1)),
                pltpu.SemaphoreType.DMA((N_PHASES, N_DEV - 1)),
            ]
        ),
    )(x, Win0, Wout0, Win1, Wout1, Win2, Wout2)


# device time: 34150 ns/iter; 1.0213x vs baseline; 1.0213x over previous
import jax
import jax.numpy as jnp
from jax import lax
from jax.experimental import pallas as pl
from jax.experimental.pallas import tpu as pltpu

N_DEV = 4
N_PHASES = 6
RECV_ORDER = (1, 3, 2)
SEND_ORDER = (2, 1, 3)


def kernel(x, Win0, Wout0, Win1, Wout1, Win2, Wout2):
    b, d = x.shape

    def body(x_ref, win0, wout0, win1, wout1, win2, wout2, out_ref,
             xbuf0, xbuf1, xbuf2, pbuf0, pbuf1, pbuf2,
             psend0, psend1, psend2, send_sems, recv_sems):
        me = lax.axis_index("i")
        xbufs = [xbuf0, xbuf1, xbuf2]
        pbufs = [pbuf0, pbuf1, pbuf2]
        psends = [psend0, psend1, psend2]
        wins = [win0, win1, win2]
        wouts = [wout0, wout1, wout2]
        sends = []

        def send(phase, idx, src, dst_buf, tgt):
            rdma = pltpu.make_async_remote_copy(
                src_ref=src,
                dst_ref=dst_buf.at[me],
                send_sem=send_sems.at[phase, idx],
                recv_sem=recv_sems.at[phase, idx],
                device_id=(tgt,),
                device_id_type=pl.DeviceIdType.MESH,
            )
            rdma.start()
            sends.append(rdma)

        def wait_recv(phase, idx, buf):
            rdma = pltpu.make_async_remote_copy(
                src_ref=buf.at[me],
                dst_ref=buf.at[me],
                send_sem=send_sems.at[phase, idx],
                recv_sem=recv_sems.at[phase, idx],
                device_id=(me,),
                device_id_type=pl.DeviceIdType.MESH,
            )
            rdma.wait_recv()

        def mlp(xc, win, wout):
            hh = jnp.maximum(jnp.dot(xc, win, preferred_element_type=jnp.float32), 0.0)
            return jnp.dot(hh, wout, preferred_element_type=jnp.float32)

        xbuf0[me] = x_ref[...]
        for off in SEND_ORDER:
            send(0, off - 1, x_ref, xbuf0, (me + off) % N_DEV)

        for l in range(3):
            win = wins[l][...]
            wout = wouts[l][...]
            gphase = 2 * l
            sphase = 2 * l + 1
            acc = mlp(xbufs[l][me], win, wout)
            for off in RECV_ORDER:
                src = (me - off) % N_DEV
                wait_recv(gphase, off - 1, xbufs[l])
                psends[l][src] = mlp(xbufs[l][src], win, wout)
                send(sphase, 3 - off, psends[l].at[src], pbufs[l], src)
            for off in RECV_ORDER:
                wait_recv(sphase, off - 1, pbufs[l])
                acc = acc + pbufs[l][(me - off) % N_DEV]
            if l < 2:
                xbufs[l + 1][me] = acc
                for off in SEND_ORDER:
                    send(2 * l + 2, off - 1, xbufs[l + 1].at[me],
                         xbufs[l + 1], (me + off) % N_DEV)
            else:
                out_ref[...] = acc

        for rdma in sends:
            rdma.wait_send()

    return pl.pallas_call(
        body,
        out_shape=jax.ShapeDtypeStruct((b, d), jnp.float32),
        in_specs=[pl.BlockSpec(memory_space=pltpu.VMEM)] * 7,
        out_specs=pl.BlockSpec(memory_space=pltpu.VMEM),
        scratch_shapes=(
            [pltpu.VMEM((N_DEV, b, d), jnp.float32)] * 3
            + [pltpu.VMEM((N_DEV, b, d), jnp.float32)] * 3
            + [pltpu.VMEM((N_DEV, b, d), jnp.float32)] * 3
            + [
                pltpu.SemaphoreType.DMA((N_PHASES, N_DEV - 1)),
                pltpu.SemaphoreType.DMA((N_PHASES, N_DEV - 1)),
            ]
        ),
    )(x, Win0, Wout0, Win1, Wout1, Win2, Wout2)
